# baseline (speedup 1.0000x reference)
import functools

import jax
import jax.numpy as jnp
from jax import lax
from jax.experimental import pallas as pl
from jax.experimental.pallas import tpu as pltpu
from jax.experimental.pallas import tpu_sc as plsc

BATCH = 16384
EMBED_DIM = 32
LANES = 16
NUM_WORKERS = 32
B_PER_W = BATCH // NUM_WORKERS
IDX_CHUNK = 128
N_CHUNKS = B_PER_W // IDX_CHUNK
GROUPS = B_PER_W // LANES


def _body(uidx_hbm, pidx_hbm, utab_hbm, ptab_hbm, out_hbm,
          uidx_v, pidx_v, urows_v, prows_v, out_v, sem):
    cid = lax.axis_index("c")
    sid = lax.axis_index("s")
    wid = sid * 2 + cid
    base = wid * B_PER_W

    pltpu.sync_copy(uidx_hbm.at[pl.ds(wid * N_CHUNKS, N_CHUNKS)], uidx_v)
    pltpu.sync_copy(pidx_hbm.at[pl.ds(wid * N_CHUNKS, N_CHUNKS)], pidx_v)

    copies = []
    for j in range(N_CHUNKS):
        copies.append(pltpu.async_copy(
            utab_hbm.at[uidx_v.at[j]], urows_v.at[pl.ds(j * IDX_CHUNK, IDX_CHUNK)], sem))
        copies.append(pltpu.async_copy(
            ptab_hbm.at[pidx_v.at[j]], prows_v.at[pl.ds(j * IDX_CHUNK, IDX_CHUNK)], sem))
    for c in copies:
        c.wait()

    lane = lax.iota(jnp.int32, 16)
    col = [jnp.full((16,), d, jnp.int32) for d in range(EMBED_DIM)]

    def group(g, _):
        rows = g * LANES + lane
        acc = jnp.zeros((16,), jnp.float32)
        for d in range(EMBED_DIM):
            uvals = plsc.load_gather(urows_v, [rows, col[d]])
            pvals = plsc.load_gather(prows_v, [rows, col[d]])
            acc = acc + uvals * pvals
        out_v[pl.ds(g * LANES, LANES)] = acc
        return 0

    lax.fori_loop(0, GROUPS, group, 0)

    pltpu.sync_copy(out_v, out_hbm.at[pl.ds(base, B_PER_W)])


@jax.jit
def _sc_dot(uidx, pidx, user_table, place_table):
    mesh = plsc.VectorSubcoreMesh(core_axis_name="c", subcore_axis_name="s")
    kern = pl.kernel(
        _body,
        out_type=jax.ShapeDtypeStruct((BATCH,), jnp.float32),
        mesh=mesh,
        scratch_types=[
            pltpu.VMEM((N_CHUNKS, IDX_CHUNK), jnp.int32),
            pltpu.VMEM((N_CHUNKS, IDX_CHUNK), jnp.int32),
            pltpu.VMEM((B_PER_W, EMBED_DIM), jnp.float32),
            pltpu.VMEM((B_PER_W, EMBED_DIM), jnp.float32),
            pltpu.VMEM((B_PER_W,), jnp.float32),
            pltpu.SemaphoreType.DMA,
        ],
        compiler_params=pltpu.CompilerParams(
            needs_layout_passes=False, use_tc_tiling_on_sc=False),
    )
    return kern(uidx, pidx, user_table, place_table)


def kernel(inputs, user_table, place_table):
    uidx = inputs[:, 0].astype(jnp.int32).reshape(-1, IDX_CHUNK)
    pidx = inputs[:, 1].astype(jnp.int32).reshape(-1, IDX_CHUNK)
    return _sc_dot(uidx, pidx, user_table, place_table)
